# Initial kernel scaffold; baseline (speedup 1.0000x reference)
import jax
import jax.numpy as jnp
from jax.experimental import pallas as pl


def kernel(x, q, k):
    raise NotImplementedError("write your pallas kernel here")


# speedup vs baseline: 6.6049x; 6.6049x over previous
import functools

import jax
import jax.numpy as jnp
from jax import lax
from jax.experimental import pallas as pl
from jax.experimental.pallas import tpu as pltpu
from jax.experimental.pallas import tpu_sc as plsc

B, C, P, Q, K = 4, 5, 8192, 2048, 16
NC, NS, L = 2, 16, 16
NW = NC * NS
WPB = NW // B
QPW = Q // WPB
NCHUNK = P // L
QB = 64


def _bf16_rne(v):
    u = lax.bitcast_convert_type(v, jnp.uint32)
    r = (u + jnp.uint32(0x7FFF) + ((u >> jnp.uint32(16)) & jnp.uint32(1)))
    r = r & jnp.uint32(0xFFFF0000)
    return lax.bitcast_convert_type(r, jnp.float32)


def _knn_body(x_hbm, q_hbm, out_hbm, xloc, xbloc, xxloc, qloc, qqloc, oloc):
    wid = lax.axis_index("s") * NC + lax.axis_index("c")
    b = wid // WPB
    qs = (wid % WPB) * QPW

    pltpu.sync_copy(x_hbm.at[b], xloc)
    for ch in range(3):
        pltpu.sync_copy(q_hbm.at[b, ch, pl.ds(qs, QPW)],
                        qloc.at[pl.ds(ch * QPW, QPW)])

    def xx_body(c2, _):
        px = xloc[0, pl.ds(c2 * L, L)]
        py = xloc[1, pl.ds(c2 * L, L)]
        pz = xloc[2, pl.ds(c2 * L, L)]
        xxloc[pl.ds(c2 * L, L)] = -(px * px + py * py + pz * pz)
        xbloc[pl.ds(0 * P + c2 * L, L)] = _bf16_rne(px)
        xbloc[pl.ds(1 * P + c2 * L, L)] = _bf16_rne(py)
        xbloc[pl.ds(2 * P + c2 * L, L)] = _bf16_rne(pz)
        return 0

    lax.fori_loop(0, NCHUNK, xx_body, 0)

    def qq_body(g, _):
        qx = qloc[pl.ds(0 * QPW + g * L, L)]
        qy = qloc[pl.ds(1 * QPW + g * L, L)]
        qz = qloc[pl.ds(2 * QPW + g * L, L)]
        qqloc[pl.ds(g * L, L)] = qx * qx + qy * qy + qz * qz
        qloc[pl.ds(0 * QPW + g * L, L)] = 2.0 * _bf16_rne(qx)
        qloc[pl.ds(1 * QPW + g * L, L)] = 2.0 * _bf16_rne(qy)
        qloc[pl.ds(2 * QPW + g * L, L)] = 2.0 * _bf16_rne(qz)
        return 0

    lax.fori_loop(0, QPW // L, qq_body, 0)

    iota = lax.iota(jnp.int32, L)
    lane0 = jnp.zeros((L,), jnp.int32)
    neg_inf = jnp.full((L,), -jnp.inf, jnp.float32)

    def blk_body(blk, _):
      def q_body(jj, _):
        j = blk * QB + jj
        g16 = (j // L) * L
        lane = jnp.full((L,), j % L, jnp.int32)
        qxv = qloc[pl.ds(0 * QPW + g16, L)].at[lane].get(mode="promise_in_bounds")
        qyv = qloc[pl.ds(1 * QPW + g16, L)].at[lane].get(mode="promise_in_bounds")
        qzv = qloc[pl.ds(2 * QPW + g16, L)].at[lane].get(mode="promise_in_bounds")
        qqv = qqloc[pl.ds(g16, L)].at[lane].get(mode="promise_in_bounds")

        def chunk_body(c2, carry):
            bv, bi, tv = carry
            base = c2 * L
            px = xbloc[pl.ds(0 * P + base, L)]
            py = xbloc[pl.ds(1 * P + base, L)]
            pz = xbloc[pl.ds(2 * P + base, L)]
            nxxv = xxloc[pl.ds(base, L)]
            s = px * qxv + py * qyv + pz * qzv
            pd = (nxxv + s) - qqv

            def do_merge(bv, bi, tv, pd):
                ci = base + iota
                sv, si = plsc.sort_key_val(pd, ci, descending=True)
                take = sv > bv
                nv = jnp.where(take, sv, bv)
                ni = jnp.where(take, si, bi)
                bv2, bi2 = plsc.sort_key_val(nv, ni)
                tv2 = bv2.at[lane0].get(mode="promise_in_bounds")
                return bv2, bi2, tv2

            cnt = plsc.all_reduce_population_count(pd > tv)
            return lax.cond(cnt[0] > 0, do_merge,
                            lambda bv, bi, tv, pd: (bv, bi, tv),
                            bv, bi, tv, pd)

        bv, bi, _ = lax.fori_loop(
            0, NCHUNK, chunk_body,
            (neg_inf, jnp.zeros((L,), jnp.int32), neg_inf))

        nbr = lax.rev(bi, (0,))
        for ch in range(C):
            fv = plsc.load_gather(
                xloc, [jnp.full((L,), ch, jnp.int32), nbr])
            oloc[ch, pl.ds(jj * K, K)] = fv
        return 0

      lax.fori_loop(0, QB, q_body, 0)
      pltpu.sync_copy(oloc, out_hbm.at[wid, :, pl.ds(blk * QB * K, QB * K)])
      return 0

    lax.fori_loop(0, QPW // QB, blk_body, 0)


@functools.partial(
    pl.kernel,
    out_type=jax.ShapeDtypeStruct((NW, C, QPW * K), jnp.float32),
    mesh=plsc.VectorSubcoreMesh(core_axis_name="c", subcore_axis_name="s"),
    compiler_params=pltpu.CompilerParams(needs_layout_passes=False),
    scratch_types=[
        pltpu.VMEM((C, P), jnp.float32),
        pltpu.VMEM((3 * P,), jnp.float32),
        pltpu.VMEM((P,), jnp.float32),
        pltpu.VMEM((3 * QPW,), jnp.float32),
        pltpu.VMEM((QPW,), jnp.float32),
        pltpu.VMEM((C, QB * K), jnp.float32),
    ],
)
def _knn_sc(x_hbm, q_hbm, out_hbm, xloc, xbloc, xxloc, qloc, qqloc, oloc):
    _knn_body(x_hbm, q_hbm, out_hbm, xloc, xbloc, xxloc, qloc, qqloc, oloc)


def kernel(x, q, k):
    del k
    out = _knn_sc(x, q)
    out = out.reshape(B, WPB, C, QPW, K).transpose(0, 2, 1, 3, 4)
    return out.reshape(B, C, Q, K)


# speedup vs baseline: 38.5517x; 5.8369x over previous
import functools

import jax
import jax.numpy as jnp
from jax import lax
from jax.experimental import pallas as pl
from jax.experimental.pallas import tpu as pltpu
from jax.experimental.pallas import tpu_sc as plsc

B, C, P, Q, K = 4, 5, 8192, 2048, 16
NC, NS, L = 2, 16, 16
NW = NC * NS
WPB = NW // B
QPW = Q // WPB
NCHUNK = P // L
QB = 64
GQ = 4


def _bf16_rne(v):
    u = lax.bitcast_convert_type(v, jnp.uint32)
    r = (u + jnp.uint32(0x7FFF) + ((u >> jnp.uint32(16)) & jnp.uint32(1)))
    r = r & jnp.uint32(0xFFFF0000)
    return lax.bitcast_convert_type(r, jnp.float32)


def _knn_body(x_hbm, q_hbm, out_hbm, xloc, xbloc, xxloc, qloc, qqloc,
              mloc, oloc):
    wid = lax.axis_index("s") * NC + lax.axis_index("c")
    b = wid // WPB
    qs = (wid % WPB) * QPW

    pltpu.sync_copy(x_hbm.at[b], xloc)
    for ch in range(3):
        pltpu.sync_copy(q_hbm.at[b, ch, pl.ds(qs, QPW)],
                        qloc.at[pl.ds(ch * QPW, QPW)])

    def xx_body(c2, _):
        px = xloc[0, pl.ds(c2 * L, L)]
        py = xloc[1, pl.ds(c2 * L, L)]
        pz = xloc[2, pl.ds(c2 * L, L)]
        xxloc[pl.ds(c2 * L, L)] = -(px * px + py * py + pz * pz)
        xbloc[pl.ds(0 * P + c2 * L, L)] = _bf16_rne(px)
        xbloc[pl.ds(1 * P + c2 * L, L)] = _bf16_rne(py)
        xbloc[pl.ds(2 * P + c2 * L, L)] = _bf16_rne(pz)
        return 0

    lax.fori_loop(0, NCHUNK, xx_body, 0)

    def qq_body(g, _):
        qx = qloc[pl.ds(0 * QPW + g * L, L)]
        qy = qloc[pl.ds(1 * QPW + g * L, L)]
        qz = qloc[pl.ds(2 * QPW + g * L, L)]
        qqloc[pl.ds(g * L, L)] = qx * qx + qy * qy + qz * qz
        qloc[pl.ds(0 * QPW + g * L, L)] = 2.0 * _bf16_rne(qx)
        qloc[pl.ds(1 * QPW + g * L, L)] = 2.0 * _bf16_rne(qy)
        qloc[pl.ds(2 * QPW + g * L, L)] = 2.0 * _bf16_rne(qz)
        return 0

    lax.fori_loop(0, QPW // L, qq_body, 0)

    iota = lax.iota(jnp.int32, L)
    mask15 = iota == 15
    zeros_i = jnp.zeros((L,), jnp.int32)
    neg_inf = jnp.full((L,), -jnp.inf, jnp.float32)

    def blk_body(blk, _):
      def grp_body(g, _):
        j0 = blk * QB + g * GQ
        g16 = (j0 // L) * L
        qbase = j0 % L
        qxg = qloc[pl.ds(0 * QPW + g16, L)]
        qyg = qloc[pl.ds(1 * QPW + g16, L)]
        qzg = qloc[pl.ds(2 * QPW + g16, L)]
        qqg = qqloc[pl.ds(g16, L)]
        qxv, qyv, qzv, qqv = [], [], [], []
        for i in range(GQ):
            lane = jnp.full((L,), qbase + i, jnp.int32)
            qxv.append(qxg.at[lane].get(mode="promise_in_bounds"))
            qyv.append(qyg.at[lane].get(mode="promise_in_bounds"))
            qzv.append(qzg.at[lane].get(mode="promise_in_bounds"))
            qqv.append(qqg.at[lane].get(mode="promise_in_bounds"))

        def a_body(c2, _):
            base = c2 * L
            px = xbloc[pl.ds(0 * P + base, L)]
            py = xbloc[pl.ds(1 * P + base, L)]
            pz = xbloc[pl.ds(2 * P + base, L)]
            nxxv = xxloc[pl.ds(base, L)]
            for i in range(GQ):
                s = px * qxv[i] + py * qyv[i] + pz * qzv[i]
                pd = (nxxv + s) - qqv[i]
                cm = plsc.cummax(pd)
                plsc.store_compressed(
                    mloc.at[pl.ds(i * NCHUNK + c2, L)], cm, mask=mask15)
            return 0

        lax.fori_loop(0, NCHUNK, a_body, 0)

        for i in range(GQ):
            def b_body(c2, carry):
                bv, bi = carry
                v = mloc[pl.ds(i * NCHUNK + c2 * L, L)]
                ci = c2 * L + iota
                sv, si = plsc.sort_key_val(v, ci, descending=True)
                take = sv > bv
                nv = jnp.where(take, sv, bv)
                ni = jnp.where(take, si, bi)
                return tuple(plsc.sort_key_val(nv, ni))

            mv, mc = lax.fori_loop(0, NCHUNK // L, b_body,
                                   (neg_inf, zeros_i))

            def c_body(ii, carry):
                bv, bi = carry
                lanev = jnp.full((L,), ii, jnp.int32)
                cb = mc.at[lanev].get(mode="promise_in_bounds")
                flat = cb * L + iota
                px = plsc.load_gather(xbloc, [flat])
                py = plsc.load_gather(xbloc, [flat + P])
                pz = plsc.load_gather(xbloc, [flat + 2 * P])
                nxxv = plsc.load_gather(xxloc, [flat])
                s = px * qxv[i] + py * qyv[i] + pz * qzv[i]
                pd = (nxxv + s) - qqv[i]
                sv, si = plsc.sort_key_val(pd, flat, descending=True)
                take = sv > bv
                nv = jnp.where(take, sv, bv)
                ni = jnp.where(take, si, bi)
                return tuple(plsc.sort_key_val(nv, ni))

            bv, bi = lax.fori_loop(0, L, c_body, (neg_inf, zeros_i))

            nbr = lax.rev(bi, (0,))
            for ch in range(C):
                fv = plsc.load_gather(
                    xloc, [jnp.full((L,), ch, jnp.int32), nbr])
                oloc[ch, pl.ds((g * GQ + i) * K, K)] = fv
        return 0

      lax.fori_loop(0, QB // GQ, grp_body, 0)
      pltpu.sync_copy(oloc, out_hbm.at[wid, :, pl.ds(blk * QB * K, QB * K)])
      return 0

    lax.fori_loop(0, QPW // QB, blk_body, 0)


@functools.partial(
    pl.kernel,
    out_type=jax.ShapeDtypeStruct((NW, C, QPW * K), jnp.float32),
    mesh=plsc.VectorSubcoreMesh(core_axis_name="c", subcore_axis_name="s"),
    compiler_params=pltpu.CompilerParams(needs_layout_passes=False),
    scratch_types=[
        pltpu.VMEM((C, P), jnp.float32),
        pltpu.VMEM((3 * P,), jnp.float32),
        pltpu.VMEM((P,), jnp.float32),
        pltpu.VMEM((3 * QPW,), jnp.float32),
        pltpu.VMEM((QPW,), jnp.float32),
        pltpu.VMEM((GQ * NCHUNK + L,), jnp.float32),
        pltpu.VMEM((C, QB * K), jnp.float32),
    ],
)
def _knn_sc(x_hbm, q_hbm, out_hbm, xloc, xbloc, xxloc, qloc, qqloc,
            mloc, oloc):
    _knn_body(x_hbm, q_hbm, out_hbm, xloc, xbloc, xxloc, qloc, qqloc,
              mloc, oloc)


def kernel(x, q, k):
    del k
    out = _knn_sc(x, q)
    out = out.reshape(B, WPB, C, QPW, K).transpose(0, 2, 1, 3, 4)
    return out.reshape(B, C, Q, K)


# speedup vs baseline: 49.7082x; 1.2894x over previous
import functools

import jax
import jax.numpy as jnp
from jax import lax
from jax.experimental import pallas as pl
from jax.experimental.pallas import tpu as pltpu
from jax.experimental.pallas import tpu_sc as plsc

B, C, P, Q, K = 4, 5, 8192, 2048, 16
NC, NS, L = 2, 16, 16
NW = NC * NS
WPB = NW // B
QPW = Q // WPB
NCHUNK = P // L
QB = 64
GQ = 8


def _bf16_rne(v):
    u = lax.bitcast_convert_type(v, jnp.uint32)
    r = (u + jnp.uint32(0x7FFF) + ((u >> jnp.uint32(16)) & jnp.uint32(1)))
    r = r & jnp.uint32(0xFFFF0000)
    return lax.bitcast_convert_type(r, jnp.float32)


def _knn_body(x_hbm, q_hbm, out_hbm, xloc, xbloc, xxloc, qloc, qqloc,
              mloc, oloc):
    wid = lax.axis_index("s") * NC + lax.axis_index("c")
    b = wid // WPB
    qs = (wid % WPB) * QPW

    pltpu.sync_copy(x_hbm.at[b], xloc)
    for ch in range(3):
        pltpu.sync_copy(q_hbm.at[b, ch, pl.ds(qs, QPW)],
                        qloc.at[pl.ds(ch * QPW, QPW)])

    def xx_body(c2, _):
        px = xloc[0, pl.ds(c2 * L, L)]
        py = xloc[1, pl.ds(c2 * L, L)]
        pz = xloc[2, pl.ds(c2 * L, L)]
        xxloc[pl.ds(c2 * L, L)] = -(px * px + py * py + pz * pz)
        xbloc[pl.ds(0 * P + c2 * L, L)] = _bf16_rne(px)
        xbloc[pl.ds(1 * P + c2 * L, L)] = _bf16_rne(py)
        xbloc[pl.ds(2 * P + c2 * L, L)] = _bf16_rne(pz)
        return 0

    lax.fori_loop(0, NCHUNK, xx_body, 0)

    def qq_body(g, _):
        qx = qloc[pl.ds(0 * QPW + g * L, L)]
        qy = qloc[pl.ds(1 * QPW + g * L, L)]
        qz = qloc[pl.ds(2 * QPW + g * L, L)]
        qqloc[pl.ds(g * L, L)] = qx * qx + qy * qy + qz * qz
        qloc[pl.ds(0 * QPW + g * L, L)] = 2.0 * _bf16_rne(qx)
        qloc[pl.ds(1 * QPW + g * L, L)] = 2.0 * _bf16_rne(qy)
        qloc[pl.ds(2 * QPW + g * L, L)] = 2.0 * _bf16_rne(qz)
        return 0

    lax.fori_loop(0, QPW // L, qq_body, 0)

    iota = lax.iota(jnp.int32, L)
    mask15 = iota == 15
    zeros_i = jnp.zeros((L,), jnp.int32)
    neg_inf = jnp.full((L,), -jnp.inf, jnp.float32)

    def blk_body(blk, _):
      def grp_body(g, _):
        j0 = blk * QB + g * GQ
        g16 = (j0 // L) * L
        qbase = j0 % L
        qxg = qloc[pl.ds(0 * QPW + g16, L)]
        qyg = qloc[pl.ds(1 * QPW + g16, L)]
        qzg = qloc[pl.ds(2 * QPW + g16, L)]
        qqg = qqloc[pl.ds(g16, L)]
        qxv, qyv, qzv, qqv = [], [], [], []
        for i in range(GQ):
            lane = jnp.full((L,), qbase + i, jnp.int32)
            qxv.append(qxg.at[lane].get(mode="promise_in_bounds"))
            qyv.append(qyg.at[lane].get(mode="promise_in_bounds"))
            qzv.append(qzg.at[lane].get(mode="promise_in_bounds"))
            qqv.append(qqg.at[lane].get(mode="promise_in_bounds"))

        def a_body(c2, _):
            base = c2 * L
            px = xbloc[pl.ds(0 * P + base, L)]
            py = xbloc[pl.ds(1 * P + base, L)]
            pz = xbloc[pl.ds(2 * P + base, L)]
            nxxv = xxloc[pl.ds(base, L)]
            for i in range(GQ):
                s = px * qxv[i] + py * qyv[i] + pz * qzv[i]
                pd = (nxxv + s) - qqv[i]
                cm = plsc.cummax(pd)
                plsc.store_compressed(
                    mloc.at[pl.ds(i * NCHUNK + c2, L)], cm, mask=mask15)
            return 0

        lax.fori_loop(0, NCHUNK, a_body, 0)

        for i in range(GQ):
            def b_body(c2, carry):
                bv, bi = carry
                v = mloc[pl.ds(i * NCHUNK + c2 * L, L)]
                ci = c2 * L + iota
                sv, si = plsc.sort_key_val(v, ci, descending=True)
                take = sv > bv
                nv = jnp.where(take, sv, bv)
                ni = jnp.where(take, si, bi)
                return tuple(plsc.sort_key_val(nv, ni))

            mv, mc = lax.fori_loop(0, NCHUNK // L, b_body,
                                   (neg_inf, zeros_i))

            def c_body(ii, carry):
                bv, bi = carry
                lanev = jnp.full((L,), ii, jnp.int32)
                cb = mc.at[lanev].get(mode="promise_in_bounds")
                flat = cb * L + iota
                px = plsc.load_gather(xbloc, [flat])
                py = plsc.load_gather(xbloc, [flat + P])
                pz = plsc.load_gather(xbloc, [flat + 2 * P])
                nxxv = plsc.load_gather(xxloc, [flat])
                s = px * qxv[i] + py * qyv[i] + pz * qzv[i]
                pd = (nxxv + s) - qqv[i]
                sv, si = plsc.sort_key_val(pd, flat, descending=True)
                take = sv > bv
                nv = jnp.where(take, sv, bv)
                ni = jnp.where(take, si, bi)
                return tuple(plsc.sort_key_val(nv, ni))

            bv, bi = lax.fori_loop(0, L, c_body, (neg_inf, zeros_i))

            nbr = lax.rev(bi, (0,))
            for ch in range(C):
                fv = plsc.load_gather(
                    xloc, [jnp.full((L,), ch, jnp.int32), nbr])
                oloc[ch, pl.ds((g * GQ + i) * K, K)] = fv
        return 0

      lax.fori_loop(0, QB // GQ, grp_body, 0)
      pltpu.sync_copy(oloc, out_hbm.at[wid, :, pl.ds(blk * QB * K, QB * K)])
      return 0

    lax.fori_loop(0, QPW // QB, blk_body, 0)


@functools.partial(
    pl.kernel,
    out_type=jax.ShapeDtypeStruct((NW, C, QPW * K), jnp.float32),
    mesh=plsc.VectorSubcoreMesh(core_axis_name="c", subcore_axis_name="s"),
    compiler_params=pltpu.CompilerParams(needs_layout_passes=False),
    scratch_types=[
        pltpu.VMEM((C, P), jnp.float32),
        pltpu.VMEM((3 * P,), jnp.float32),
        pltpu.VMEM((P,), jnp.float32),
        pltpu.VMEM((3 * QPW,), jnp.float32),
        pltpu.VMEM((QPW,), jnp.float32),
        pltpu.VMEM((GQ * NCHUNK + L,), jnp.float32),
        pltpu.VMEM((C, QB * K), jnp.float32),
    ],
)
def _knn_sc(x_hbm, q_hbm, out_hbm, xloc, xbloc, xxloc, qloc, qqloc,
            mloc, oloc):
    _knn_body(x_hbm, q_hbm, out_hbm, xloc, xbloc, xxloc, qloc, qqloc,
              mloc, oloc)


def kernel(x, q, k):
    del k
    out = _knn_sc(x, q)
    out = out.reshape(B, WPB, C, QPW, K).transpose(0, 2, 1, 3, 4)
    return out.reshape(B, C, Q, K)
